# Initial kernel scaffold; baseline (speedup 1.0000x reference)
import jax
import jax.numpy as jnp
from jax.experimental import pallas as pl


def kernel(x, edge_index, ln_gamma, ln_beta, W_l, b_l, W_r):
    raise NotImplementedError("write your pallas kernel here")


# speedup vs baseline: 4.2644x; 4.2644x over previous
import functools

import jax
import jax.numpy as jnp
from jax import lax
from jax.experimental import pallas as pl
from jax.experimental.pallas import tpu as pltpu
from jax.experimental.pallas import tpu_sc as plsc

NC = 2
NS = 16
NW = NC * NS
CH = 128


def _ln_relu_body(x_ref, g_ref, b_ref, o_ref):
    xb = x_ref[...]
    mean = jnp.mean(xb, axis=-1, keepdims=True)
    var = jnp.mean((xb - mean) ** 2, axis=-1, keepdims=True)
    h = (xb - mean) * lax.rsqrt(var + 1e-5) * g_ref[...] + b_ref[...]
    o_ref[...] = jnp.maximum(h, 0.0)


def _ln_relu(x, gamma, beta, blk):
    n, d = x.shape
    return pl.pallas_call(
        _ln_relu_body,
        grid=(n // blk,),
        in_specs=[
            pl.BlockSpec((blk, d), lambda i: (i, 0)),
            pl.BlockSpec((1, d), lambda i: (0, 0)),
            pl.BlockSpec((1, d), lambda i: (0, 0)),
        ],
        out_specs=pl.BlockSpec((blk, d), lambda i: (i, 0)),
        out_shape=jax.ShapeDtypeStruct((n, d), jnp.float32),
    )(x, gamma.reshape(1, d), beta.reshape(1, d))


def _make_sc_agg(n, d, ntab, n_chunks, per_w):
    mesh = plsc.VectorSubcoreMesh(
        core_axis_name="c", subcore_axis_name="s", num_cores=NC, num_subcores=NS
    )
    zstripe = ntab // NS

    @functools.partial(
        pl.kernel,
        out_type=(
            jax.ShapeDtypeStruct((NC, ntab, d), jnp.float32),
            jax.ShapeDtypeStruct((NC, ntab, d), jnp.float32),
        ),
        mesh=mesh,
        scratch_types=[
            pltpu.VMEM_SHARED((ntab, d), jnp.float32),
            pltpu.VMEM((CH,), jnp.int32),
            pltpu.VMEM((CH,), jnp.int32),
            pltpu.VMEM((CH, d), jnp.float32),
            pltpu.SemaphoreType.DMA,
        ],
    )
    def sc_agg(h_hbm, src_hbm, dst_hbm,
               agg_out, deg_out, tab_sh, sidx, didx, rows, sem):
        c = lax.axis_index("c")
        s = lax.axis_index("s")
        wid = c * NS + s
        z0 = s * zstripe

        lane = lax.iota(jnp.int32, 16)
        zv = jnp.where(lane < 0, jnp.float32(1.0), jnp.float32(0.0))
        ov = jnp.where(lane >= 0, jnp.float32(1.0), jnp.float32(0.0))

        def fill_rows(val):
            def body(i, carry):
                for j in range(d // 16):
                    rows[i, pl.ds(j * 16, 16)] = val
                return carry
            lax.fori_loop(0, CH, body, 0)

        def zero_table():
            for t in range(zstripe // CH):
                pltpu.sync_copy(rows, tab_sh.at[pl.ds(z0 + t * CH, CH)])

        def copy_table(out3):
            pltpu.sync_copy(tab_sh.at[pl.ds(z0, zstripe)],
                            out3.at[c, pl.ds(z0, zstripe)])

        fill_rows(zv)
        zero_table()
        plsc.subcore_barrier()

        def step_a(k, carry):
            base = pl.multiple_of(wid * per_w + k * CH, CH)
            pltpu.sync_copy(src_hbm.at[pl.ds(base, CH)], sidx)
            pltpu.sync_copy(dst_hbm.at[pl.ds(base, CH)], didx)
            pltpu.async_copy(h_hbm.at[sidx], rows, sem).wait()
            pltpu.sync_copy(rows, tab_sh.at[didx], add=True)
            return carry

        lax.fori_loop(0, n_chunks, step_a, 0)
        plsc.subcore_barrier()
        copy_table(agg_out)
        plsc.subcore_barrier()

        fill_rows(zv)
        zero_table()
        plsc.subcore_barrier()
        fill_rows(ov)

        def step_b(k, carry):
            base = pl.multiple_of(wid * per_w + k * CH, CH)
            pltpu.sync_copy(dst_hbm.at[pl.ds(base, CH)], didx)
            pltpu.sync_copy(rows, tab_sh.at[didx], add=True)
            return carry

        lax.fori_loop(0, n_chunks, step_b, 0)
        plsc.subcore_barrier()
        copy_table(deg_out)

    return sc_agg


def _final_body(x_ref, h_ref, a0_ref, a1_ref, d0_ref, d1_ref,
                wl_ref, wr_ref, bl_ref, o_ref):
    agg = a0_ref[...] + a1_ref[...]
    deg = jnp.maximum(d0_ref[...] + d1_ref[...], 1.0)
    am = agg / deg
    acc = jnp.dot(am, wl_ref[...], preferred_element_type=jnp.float32)
    acc += jnp.dot(h_ref[...], wr_ref[...], preferred_element_type=jnp.float32)
    o_ref[...] = acc + bl_ref[...] + x_ref[...]


def _final(x, h, agg_parts, deg_parts, w_l_t, w_r_t, b_l, blk):
    n, d = x.shape
    row = lambda i: (i, 0)
    full = lambda i: (0, 0)
    return pl.pallas_call(
        _final_body,
        grid=(n // blk,),
        in_specs=[
            pl.BlockSpec((blk, d), row),
            pl.BlockSpec((blk, d), row),
            pl.BlockSpec((blk, d), row),
            pl.BlockSpec((blk, d), row),
            pl.BlockSpec((blk, d), row),
            pl.BlockSpec((blk, d), row),
            pl.BlockSpec((d, d), full),
            pl.BlockSpec((d, d), full),
            pl.BlockSpec((1, d), full),
        ],
        out_specs=pl.BlockSpec((blk, d), row),
        out_shape=jax.ShapeDtypeStruct((n, d), jnp.float32),
    )(x, h, agg_parts[0], agg_parts[1], deg_parts[0], deg_parts[1],
      w_l_t, w_r_t, b_l.reshape(1, d))


def kernel(x, edge_index, ln_gamma, ln_beta, W_l, b_l, W_r):
    n, d = x.shape
    e = edge_index.shape[1]

    n_chunks = -(-e // (NW * CH))
    e_pad = n_chunks * NW * CH
    per_w = n_chunks * CH
    src = edge_index[0].astype(jnp.int32)
    dst = edge_index[1].astype(jnp.int32)
    pad = e_pad - e
    if pad:
        src = jnp.concatenate([src, jnp.zeros((pad,), jnp.int32)])
        dst = jnp.concatenate([dst, jnp.full((pad,), n, jnp.int32)])

    ntab = -(-(n + 1) // (NS * CH)) * NS * CH

    h = _ln_relu(x, ln_gamma, ln_beta, blk=1000)
    agg_parts, deg_parts = _make_sc_agg(n, d, ntab, n_chunks, per_w)(
        h, src, dst)
    return _final(x, h, agg_parts, deg_parts,
                  W_l.T, W_r.T, b_l, blk=1000)
